# baseline (speedup 1.0000x reference)
import functools

import jax
import jax.numpy as jnp
from jax import lax
from jax.experimental import pallas as pl
from jax.experimental.pallas import tpu as pltpu
from jax.experimental.pallas import tpu_sc as plsc

N_NODES = 50000
N_EDGES = 1600000
CUTOFF = 12.0
CUTOFF_SQ = CUTOFF * CUTOFF
INV_CSQ = 1.0 / CUTOFF_SQ
TWO_OVER_C = 2.0 / CUTOFF
INV_CSR = 1.0 / 2.0
KEHALF = 7.199822675975274

NW = 32
EDGES_PER_TILE = N_EDGES // NW
CHUNK = 2000
N_CHUNKS = EDGES_PER_TILE // CHUNK
STEPS = CHUNK // 16
ACC_ROWS = 392
ACC_N = ACC_ROWS * 128


def _tile_body(chg_hbm, dist_hbm, ii_hbm, jj_hbm, out_hbm,
               chg_v, acc_v, d0, d1, i0, i1, j0, j1, csem, sem0, sem1):
    wid = lax.axis_index("s") * 2 + lax.axis_index("c")

    ebase = wid * EDGES_PER_TILE
    dbufs, ibufs, jbufs = (d0, d1), (i0, i1), (j0, j1)
    sems = (sem0, sem1)

    def issue(c, b):
        off = ebase + c * CHUNK
        pltpu.async_copy(dist_hbm.at[pl.ds(off, CHUNK)], dbufs[b], sems[b])
        pltpu.async_copy(ii_hbm.at[pl.ds(off, CHUNK)], ibufs[b], sems[b])
        pltpu.async_copy(jj_hbm.at[pl.ds(off, CHUNK)], jbufs[b], sems[b])

    cp = pltpu.async_copy(chg_hbm, chg_v, csem)
    issue(0, 0)

    @plsc.parallel_loop(0, ACC_N // 16, step=1, unroll=4)
    def _zrow(i):
        acc_v[pl.ds(i * 16, 16)] = jnp.zeros((16,), jnp.float32)
    cp.wait()

    def drain(b):
        pltpu.make_async_copy(dist_hbm.at[pl.ds(0, CHUNK)], dbufs[b], sems[b]).wait()
        pltpu.make_async_copy(ii_hbm.at[pl.ds(0, CHUNK)], ibufs[b], sems[b]).wait()
        pltpu.make_async_copy(jj_hbm.at[pl.ds(0, CHUNK)], jbufs[b], sems[b]).wait()

    def compute(b):
        d_v, i_v, j_v = dbufs[b], ibufs[b], jbufs[b]

        @plsc.parallel_loop(0, STEPS, step=1, unroll=8)
        def _step(s):
            o = s * 16
            d = d_v[pl.ds(o, 16)]
            ii = i_v[pl.ds(o, 16)]
            jj = j_v[pl.ds(o, 16)]
            qi = plsc.load_gather(chg_v, [ii])
            qj = plsc.load_gather(chg_v, [jj])

            d2 = d * d
            sh = d2 + 1.0
            bits = plsc.bitcast(sh, jnp.int32)
            g = jnp.int32(0x5F3759DF) - lax.shift_right_logical(bits, 1)
            y = plsc.bitcast(g, jnp.float32)
            nh = sh * (-0.5)
            y = y * (1.5 + nh * y * y)
            ds = sh * y

            inv_d = 1.0 / d
            e_ord = inv_d + d * INV_CSQ - TWO_OVER_C
            e_shl = y + ds * INV_CSQ - TWO_OVER_C

            x = jnp.minimum(d * INV_CSR, 1.0)
            x2 = x * x
            x3 = x2 * x
            sw_off = 1.0 - x3 * (10.0 - 15.0 * x + 6.0 * x2)

            e = qi * qj * (e_ord + sw_off * (e_shl - e_ord))
            msk = d <= CUTOFF
            plsc.addupdate_scatter(acc_v, [ii], e, mask=msk)

    issue(0, 0)

    def pair(p, carry):
        issue(2 * p + 1, 1)
        drain(0)
        compute(0)
        issue(2 * p + 2, 0)
        drain(1)
        compute(1)
        return carry
    lax.fori_loop(0, (N_CHUNKS - 1) // 2, pair, 0)
    drain(0)
    compute(0)

    pltpu.sync_copy(acc_v, out_hbm.at[wid])


_sc_partials = functools.partial(
    pl.kernel,
    out_type=jax.ShapeDtypeStruct((NW, ACC_N), jnp.float32),
    mesh=plsc.VectorSubcoreMesh(core_axis_name="c", subcore_axis_name="s"),
    compiler_params=pltpu.CompilerParams(
        needs_layout_passes=False, use_tc_tiling_on_sc=False),
    scratch_types=[
        pltpu.VMEM((N_NODES,), jnp.float32),
        pltpu.VMEM((ACC_N,), jnp.float32),
        pltpu.VMEM((CHUNK,), jnp.float32),
        pltpu.VMEM((CHUNK,), jnp.float32),
        pltpu.VMEM((CHUNK,), jnp.int32),
        pltpu.VMEM((CHUNK,), jnp.int32),
        pltpu.VMEM((CHUNK,), jnp.int32),
        pltpu.VMEM((CHUNK,), jnp.int32),
        pltpu.SemaphoreType.DMA,
        pltpu.SemaphoreType.DMA,
        pltpu.SemaphoreType.DMA,
    ],
)(_tile_body)


def _reduce_body(p_ref, o_ref):
    o_ref[...] = KEHALF * jnp.sum(p_ref[...], axis=0)


def kernel(atomic_charges, distances, idx_i, idx_j):
    partials = _sc_partials(atomic_charges, distances, idx_i, idx_j)
    out = pl.pallas_call(
        _reduce_body,
        out_shape=jax.ShapeDtypeStruct((ACC_ROWS, 128), jnp.float32),
    )(partials.reshape(NW, ACC_ROWS, 128))
    return out.reshape(-1)[:N_NODES]


# speedup vs baseline: 1.0447x; 1.0447x over previous
import functools

import jax
import jax.numpy as jnp
from jax import lax
from jax.experimental import pallas as pl
from jax.experimental.pallas import tpu as pltpu
from jax.experimental.pallas import tpu_sc as plsc

N_NODES = 50000
N_EDGES = 1600000
CUTOFF = 12.0
CUTOFF_SQ = CUTOFF * CUTOFF
INV_CSQ = 1.0 / CUTOFF_SQ
TWO_OVER_C = 2.0 / CUTOFF
INV_CSR = 1.0 / 2.0
KEHALF = 7.199822675975274

NW = 32
EDGES_PER_TILE = N_EDGES // NW
CHUNK = 2000
N_CHUNKS = EDGES_PER_TILE // CHUNK
STEPS = CHUNK // 16
ACC_ROWS = 392
ACC_N = ACC_ROWS * 128


def _tile_body(chg_hbm, dist_hbm, ii_hbm, jj_hbm, out_hbm,
               chg_v, acc_v, d0, d1, i0, i1, j0, j1, csem, sem0, sem1):
    wid = lax.axis_index("s") * 2 + lax.axis_index("c")

    ebase = wid * EDGES_PER_TILE
    dbufs, ibufs, jbufs = (d0, d1), (i0, i1), (j0, j1)
    sems = (sem0, sem1)

    def issue(c, b):
        off = ebase + c * CHUNK
        pltpu.async_copy(dist_hbm.at[pl.ds(off, CHUNK)], dbufs[b], sems[b])
        pltpu.async_copy(ii_hbm.at[pl.ds(off, CHUNK)], ibufs[b], sems[b])
        pltpu.async_copy(jj_hbm.at[pl.ds(off, CHUNK)], jbufs[b], sems[b])

    cp = pltpu.async_copy(chg_hbm, chg_v, csem)
    issue(0, 0)

    @plsc.parallel_loop(0, ACC_N // 16, step=1, unroll=4)
    def _zrow(i):
        acc_v[pl.ds(i * 16, 16)] = jnp.zeros((16,), jnp.float32)
    cp.wait()

    def drain(b):
        pltpu.make_async_copy(dist_hbm.at[pl.ds(0, CHUNK)], dbufs[b], sems[b]).wait()
        pltpu.make_async_copy(ii_hbm.at[pl.ds(0, CHUNK)], ibufs[b], sems[b]).wait()
        pltpu.make_async_copy(jj_hbm.at[pl.ds(0, CHUNK)], jbufs[b], sems[b]).wait()

    def compute(b):
        d_v, i_v, j_v = dbufs[b], ibufs[b], jbufs[b]

        @plsc.parallel_loop(0, STEPS, step=1, unroll=4)
        def _step(s):
            o = s * 16
            d = d_v[pl.ds(o, 16)]
            ii = i_v[pl.ds(o, 16)]
            jj = j_v[pl.ds(o, 16)]
            qi = plsc.load_gather(chg_v, [ii])
            qj = plsc.load_gather(chg_v, [jj])

            d2 = d * d
            sh = d2 + 1.0
            bits = plsc.bitcast(sh, jnp.int32)
            g = jnp.int32(0x5F3759DF) - lax.shift_right_logical(bits, 1)
            y = plsc.bitcast(g, jnp.float32)
            nh = sh * (-0.5)
            y = y * (1.5 + nh * y * y)
            y = y * (1.5 + nh * y * y)
            ds = sh * y

            inv_d = 1.0 / d
            e_ord = inv_d + d * INV_CSQ - TWO_OVER_C
            e_shl = y + ds * INV_CSQ - TWO_OVER_C

            x = jnp.minimum(d * INV_CSR, 1.0)
            x2 = x * x
            x3 = x2 * x
            sw_off = 1.0 - x3 * (10.0 - 15.0 * x + 6.0 * x2)

            e = qi * qj * (e_ord + sw_off * (e_shl - e_ord))
            msk = d <= CUTOFF
            plsc.addupdate_scatter(acc_v, [ii], e, mask=msk)

    issue(0, 0)

    def pair(p, carry):
        issue(2 * p + 1, 1)
        drain(0)
        compute(0)
        issue(2 * p + 2, 0)
        drain(1)
        compute(1)
        return carry
    lax.fori_loop(0, (N_CHUNKS - 1) // 2, pair, 0)
    drain(0)
    compute(0)

    pltpu.sync_copy(acc_v, out_hbm.at[wid])


_sc_partials = functools.partial(
    pl.kernel,
    out_type=jax.ShapeDtypeStruct((NW, ACC_N), jnp.float32),
    mesh=plsc.VectorSubcoreMesh(core_axis_name="c", subcore_axis_name="s"),
    compiler_params=pltpu.CompilerParams(
        needs_layout_passes=False, use_tc_tiling_on_sc=False),
    scratch_types=[
        pltpu.VMEM((N_NODES,), jnp.float32),
        pltpu.VMEM((ACC_N,), jnp.float32),
        pltpu.VMEM((CHUNK,), jnp.float32),
        pltpu.VMEM((CHUNK,), jnp.float32),
        pltpu.VMEM((CHUNK,), jnp.int32),
        pltpu.VMEM((CHUNK,), jnp.int32),
        pltpu.VMEM((CHUNK,), jnp.int32),
        pltpu.VMEM((CHUNK,), jnp.int32),
        pltpu.SemaphoreType.DMA,
        pltpu.SemaphoreType.DMA,
        pltpu.SemaphoreType.DMA,
    ],
)(_tile_body)


def _reduce_body(p_ref, o_ref):
    o_ref[...] = KEHALF * jnp.sum(p_ref[...], axis=0)


def kernel(atomic_charges, distances, idx_i, idx_j):
    partials = _sc_partials(atomic_charges, distances, idx_i, idx_j)
    out = pl.pallas_call(
        _reduce_body,
        out_shape=jax.ShapeDtypeStruct((ACC_ROWS, 128), jnp.float32),
    )(partials.reshape(NW, ACC_ROWS, 128))
    return out.reshape(-1)[:N_NODES]
